# baseline (speedup 1.0000x reference)
import functools

import jax
import jax.numpy as jnp
from jax import lax
from jax.experimental import pallas as pl
from jax.experimental.pallas import tpu as pltpu
from jax.experimental.pallas import tpu_sc as plsc

_K = 8192
_D = 256
_B = 1024
_L = 16
_DECAY = 0.9
_THRESH = 1e-4

_BB = 512
_BH = _B // 2

_SC_NUM_CORES = 2
_SC_NUM_SUBCORES = 16


def _tc_step_kernel(decay, apply_update, res_ref, rec_ref, row_ref, w_ref,
                    cb_ref, best_ref, sidx_ref, w_out_ref, res_out_ref,
                    rec_out_ref):
    residual = res_ref[...]
    recon = rec_ref[...]
    if apply_update:
        contribution = w_ref[...] * row_ref[...]
        residual = residual - contribution
        recon = recon + contribution
    rn = jnp.sqrt(jnp.sum(residual * residual, axis=1, keepdims=True))
    active = (rn > _THRESH).astype(jnp.float32)
    rnorm = residual / jnp.maximum(rn, 1e-8)
    cos = lax.dot_general(
        rnorm.astype(jnp.bfloat16), cb_ref[...], (((1,), (1,)), ((), ())),
        preferred_element_type=jnp.float32,
        precision=lax.Precision.DEFAULT)
    maxpos = jnp.max(cos, axis=1)
    minneg = jnp.min(cos, axis=1)
    ipos = jnp.argmax(cos, axis=1).astype(jnp.int32)
    ineg = jnp.argmin(cos, axis=1).astype(jnp.int32)
    pos_wins = (maxpos > -minneg) | ((maxpos == -minneg) & (ipos < ineg))
    best = jnp.where(pos_wins, ipos, ineg)
    sign = jnp.where(pos_wins, 1.0, -1.0)
    signed_idx = jnp.where(pos_wins, best, -(best + 1))
    w = (active[:, 0] * sign) * decay
    best_ref[0, :] = best
    sidx_ref[0, :] = signed_idx
    w_out_ref[...] = w[:, None]
    res_out_ref[...] = residual
    rec_out_ref[...] = recon


def _tc_step(decay, apply_update, residual, recon, rows, w, cb_bf16):
    row_spec = pl.BlockSpec((_BB, _D), lambda i: (i, 0))
    kern = functools.partial(_tc_step_kernel, decay, apply_update)
    return pl.pallas_call(
        kern,
        grid=(_BH // _BB,),
        in_specs=[
            row_spec,
            row_spec,
            row_spec,
            pl.BlockSpec((_BB, 1), lambda i: (i, 0)),
            pl.BlockSpec((_K, _D), lambda i: (0, 0)),
        ],
        out_specs=[
            pl.BlockSpec((1, _BB), lambda i: (0, i)),
            pl.BlockSpec((1, _BB), lambda i: (0, i)),
            pl.BlockSpec((_BB, 1), lambda i: (i, 0)),
            row_spec,
            row_spec,
        ],
        out_shape=[
            jax.ShapeDtypeStruct((1, _BH), jnp.int32),
            jax.ShapeDtypeStruct((1, _BH), jnp.int32),
            jax.ShapeDtypeStruct((_BH, 1), jnp.float32),
            jax.ShapeDtypeStruct((_BH, _D), jnp.float32),
            jax.ShapeDtypeStruct((_BH, _D), jnp.float32),
        ],
    )(residual, recon, rows, w, cb_bf16)


def _tc_final_kernel(rec_ref, row_ref, w_ref, rec_out_ref):
    rec_out_ref[...] = rec_ref[...] + w_ref[...] * row_ref[...]


def _tc_final(recon, rows, w):
    row_spec = pl.BlockSpec((_BB, _D), lambda i: (i, 0))
    return pl.pallas_call(
        _tc_final_kernel,
        grid=(_BH // _BB,),
        in_specs=[
            row_spec, row_spec,
            pl.BlockSpec((_BB, 1), lambda i: (i, 0)),
        ],
        out_specs=row_spec,
        out_shape=jax.ShapeDtypeStruct((_BH, _D), jnp.float32),
    )(recon, rows, w)


@functools.cache
def _make_sc_gather():
    nw = _SC_NUM_CORES * _SC_NUM_SUBCORES
    b_per_w = _BH // nw
    mesh = plsc.VectorSubcoreMesh(core_axis_name="c", subcore_axis_name="s",
                                  num_cores=_SC_NUM_CORES)

    @functools.partial(
        pl.kernel, mesh=mesh,
        out_type=jax.ShapeDtypeStruct((_BH, _D), jnp.float32),
        scratch_types=[
            pltpu.VMEM((b_per_w,), jnp.int32),
            pltpu.VMEM((b_per_w, _D), jnp.float32),
            pltpu.SemaphoreType.DMA,
        ],
    )
    def gather(table_hbm, idx_hbm, out_hbm, idx_v, rows_v, sem):
        wid = lax.axis_index("s") * _SC_NUM_CORES + lax.axis_index("c")
        base = wid * b_per_w
        pltpu.sync_copy(idx_hbm.at[pl.ds(base, b_per_w)], idx_v)
        pltpu.async_copy(table_hbm.at[idx_v], rows_v, sem).wait()
        pltpu.sync_copy(rows_v, out_hbm.at[pl.ds(base, b_per_w)])

    return gather


def _sc_gather(table, idx):
    return _make_sc_gather()(table, idx)


@jax.jit
def kernel(targets, codebook):
    cb_bf16 = codebook.astype(jnp.bfloat16)
    halves = []
    for h in range(2):
        halves.append({
            "residual": targets[h * _BH:(h + 1) * _BH],
            "recon": jnp.zeros((_BH, _D), jnp.float32),
            "rows": jnp.zeros((_BH, _D), jnp.float32),
            "w": jnp.zeros((_BH, 1), jnp.float32),
            "idx_steps": [],
        })
    for step in range(_L):
        decay = _DECAY ** (step + 1)
        for st in halves:
            best, signed_idx, st["w"], st["residual"], st["recon"] = _tc_step(
                decay, step > 0, st["residual"], st["recon"], st["rows"],
                st["w"], cb_bf16)
            st["idx_steps"].append(signed_idx[0])
            st["rows"] = _sc_gather(codebook, best[0])
    recons = [_tc_final(st["recon"], st["rows"], st["w"]) for st in halves]
    recon = jnp.concatenate(recons, axis=0)
    signed_indices = jnp.concatenate(
        [jnp.stack(st["idx_steps"], axis=1) for st in halves], axis=0)
    return signed_indices, recon


# speedup vs baseline: 1.0414x; 1.0414x over previous
import functools

import jax
import jax.numpy as jnp
from jax import lax
from jax.experimental import pallas as pl
from jax.experimental.pallas import tpu as pltpu
from jax.experimental.pallas import tpu_sc as plsc

_K = 8192
_D = 256
_B = 1024
_L = 16
_DECAY = 0.9
_THRESH = 1e-4

_BB = 256
_BH = _B // 2

_SC_NUM_CORES = 2
_SC_NUM_SUBCORES = 16


def _tc_step_kernel(decay, apply_update, res_ref, rec_ref, row_ref, w_ref,
                    cb_ref, best_ref, sidx_ref, w_out_ref, res_out_ref,
                    rec_out_ref):
    residual = res_ref[...]
    recon = rec_ref[...]
    if apply_update:
        contribution = w_ref[...] * row_ref[...]
        residual = residual - contribution
        recon = recon + contribution
    rn = jnp.sqrt(jnp.sum(residual * residual, axis=1, keepdims=True))
    active = (rn > _THRESH).astype(jnp.float32)
    rnorm = residual / jnp.maximum(rn, 1e-8)
    cos = lax.dot_general(
        rnorm.astype(jnp.bfloat16), cb_ref[...], (((1,), (1,)), ((), ())),
        preferred_element_type=jnp.float32,
        precision=lax.Precision.DEFAULT)
    best = jnp.argmax(jnp.abs(cos), axis=1).astype(jnp.int32)
    maxpos = jnp.max(cos, axis=1)
    minneg = jnp.min(cos, axis=1)
    m = jnp.maximum(maxpos, -minneg)
    pos_fast = maxpos == m
    amb = jnp.any(pos_fast & (minneg == -m) & (m > 0))

    def _repair(_):
        ipos = jnp.argmax(cos, axis=1).astype(jnp.int32)
        ineg = jnp.argmin(cos, axis=1).astype(jnp.int32)
        two_sided = pos_fast & (minneg == -m) & (m > 0)
        win = (two_sided & (ipos < ineg)) | (~two_sided & pos_fast)
        return win.astype(jnp.int32)

    pos_wins = lax.cond(amb, _repair,
                        lambda _: pos_fast.astype(jnp.int32), None) == 1
    sign = jnp.where(pos_wins, 1.0, -1.0)
    signed_idx = jnp.where(pos_wins, best, -(best + 1))
    w = (active[:, 0] * sign) * decay
    best_ref[0, :] = best
    sidx_ref[0, :] = signed_idx
    w_out_ref[...] = w[:, None]
    res_out_ref[...] = residual
    rec_out_ref[...] = recon


def _tc_step(decay, apply_update, residual, recon, rows, w, cb_bf16):
    row_spec = pl.BlockSpec((_BB, _D), lambda i: (i, 0))
    kern = functools.partial(_tc_step_kernel, decay, apply_update)
    return pl.pallas_call(
        kern,
        grid=(_BH // _BB,),
        in_specs=[
            row_spec,
            row_spec,
            row_spec,
            pl.BlockSpec((_BB, 1), lambda i: (i, 0)),
            pl.BlockSpec((_K, _D), lambda i: (0, 0)),
        ],
        out_specs=[
            pl.BlockSpec((1, _BB), lambda i: (0, i)),
            pl.BlockSpec((1, _BB), lambda i: (0, i)),
            pl.BlockSpec((_BB, 1), lambda i: (i, 0)),
            row_spec,
            row_spec,
        ],
        out_shape=[
            jax.ShapeDtypeStruct((1, _BH), jnp.int32),
            jax.ShapeDtypeStruct((1, _BH), jnp.int32),
            jax.ShapeDtypeStruct((_BH, 1), jnp.float32),
            jax.ShapeDtypeStruct((_BH, _D), jnp.float32),
            jax.ShapeDtypeStruct((_BH, _D), jnp.float32),
        ],
    )(residual, recon, rows, w, cb_bf16)


def _tc_final_kernel(rec_ref, row_ref, w_ref, rec_out_ref):
    rec_out_ref[...] = rec_ref[...] + w_ref[...] * row_ref[...]


def _tc_final(recon, rows, w):
    row_spec = pl.BlockSpec((_BB, _D), lambda i: (i, 0))
    return pl.pallas_call(
        _tc_final_kernel,
        grid=(_BH // _BB,),
        in_specs=[
            row_spec, row_spec,
            pl.BlockSpec((_BB, 1), lambda i: (i, 0)),
        ],
        out_specs=row_spec,
        out_shape=jax.ShapeDtypeStruct((_BH, _D), jnp.float32),
    )(recon, rows, w)


@functools.cache
def _make_sc_gather():
    nw = _SC_NUM_CORES * _SC_NUM_SUBCORES
    b_per_w = _BH // nw
    mesh = plsc.VectorSubcoreMesh(core_axis_name="c", subcore_axis_name="s",
                                  num_cores=_SC_NUM_CORES)

    @functools.partial(
        pl.kernel, mesh=mesh,
        out_type=jax.ShapeDtypeStruct((_BH, _D), jnp.float32),
        scratch_types=[
            pltpu.VMEM((b_per_w,), jnp.int32),
            pltpu.VMEM((b_per_w, _D), jnp.float32),
            pltpu.SemaphoreType.DMA,
        ],
    )
    def gather(table_hbm, idx_hbm, out_hbm, idx_v, rows_v, sem):
        wid = lax.axis_index("s") * _SC_NUM_CORES + lax.axis_index("c")
        base = wid * b_per_w
        pltpu.sync_copy(idx_hbm.at[pl.ds(base, b_per_w)], idx_v)
        pltpu.async_copy(table_hbm.at[idx_v], rows_v, sem).wait()
        pltpu.sync_copy(rows_v, out_hbm.at[pl.ds(base, b_per_w)])

    return gather


def _sc_gather(table, idx):
    return _make_sc_gather()(table, idx)


@jax.jit
def kernel(targets, codebook):
    cb_bf16 = codebook.astype(jnp.bfloat16)
    halves = []
    for h in range(2):
        halves.append({
            "residual": targets[h * _BH:(h + 1) * _BH],
            "recon": jnp.zeros((_BH, _D), jnp.float32),
            "rows": jnp.zeros((_BH, _D), jnp.float32),
            "w": jnp.zeros((_BH, 1), jnp.float32),
            "idx_steps": [],
        })
    for step in range(_L):
        decay = _DECAY ** (step + 1)
        for st in halves:
            best, signed_idx, st["w"], st["residual"], st["recon"] = _tc_step(
                decay, step > 0, st["residual"], st["recon"], st["rows"],
                st["w"], cb_bf16)
            st["idx_steps"].append(signed_idx[0])
            st["rows"] = _sc_gather(codebook, best[0])
    recons = [_tc_final(st["recon"], st["rows"], st["w"]) for st in halves]
    recon = jnp.concatenate(recons, axis=0)
    signed_indices = jnp.concatenate(
        [jnp.stack(st["idx_steps"], axis=1) for st in halves], axis=0)
    return signed_indices, recon


# speedup vs baseline: 1.3078x; 1.2558x over previous
import functools

import jax
import jax.numpy as jnp
from jax import lax
from jax.experimental import pallas as pl
from jax.experimental.pallas import tpu as pltpu
from jax.experimental.pallas import tpu_sc as plsc

_K = 8192
_D = 256
_B = 1024
_L = 16
_DECAY = 0.9
_THRESH = 1e-4

_BB = 256
_BH = _B // 2

_SC_NUM_CORES = 2
_SC_NUM_SUBCORES = 16


def _finish_prev(decay_prev, res_ref, rec_ref, row_ref, act_ref, best_ref):
    residual = res_ref[...]
    recon = rec_ref[...]
    rows = row_ref[...]
    d = jnp.sum(residual * rows, axis=1)
    sign = jnp.where(d >= 0, 1.0, -1.0)
    bestp = best_ref[0, :]
    sidx = jnp.where(d >= 0, bestp, -(bestp + 1))
    w = (act_ref[..., 0] * sign) * decay_prev
    contribution = w[:, None] * rows
    return residual - contribution, recon + contribution, sidx


def _tc_step_kernel(decay_prev, apply_update, res_ref, rec_ref, row_ref,
                    act_ref, bestp_ref, cb_ref, best_ref, sidx_ref, act_out_ref,
                    res_out_ref, rec_out_ref):
    if apply_update:
        residual, recon, sidx = _finish_prev(
            decay_prev, res_ref, rec_ref, row_ref, act_ref, bestp_ref)
        sidx_ref[0, :] = sidx
    else:
        residual = res_ref[...]
        recon = rec_ref[...]
        sidx_ref[0, :] = jnp.zeros((res_ref.shape[0],), jnp.int32)
    rn = jnp.sqrt(jnp.sum(residual * residual, axis=1, keepdims=True))
    active = (rn > _THRESH).astype(jnp.float32)
    rnorm = residual / jnp.maximum(rn, 1e-8)
    cos = lax.dot_general(
        rnorm.astype(jnp.bfloat16), cb_ref[...], (((1,), (1,)), ((), ())),
        preferred_element_type=jnp.float32,
        precision=lax.Precision.DEFAULT)
    best = jnp.argmax(jnp.abs(cos), axis=1).astype(jnp.int32)
    best_ref[0, :] = best
    act_out_ref[...] = active
    res_out_ref[...] = residual
    rec_out_ref[...] = recon


def _tc_step(decay_prev, apply_update, residual, recon, rows, act, bestp,
             cb_bf16):
    row_spec = pl.BlockSpec((_BB, _D), lambda i: (i, 0))
    col_spec = pl.BlockSpec((_BB, 1), lambda i: (i, 0))
    idx_spec = pl.BlockSpec((1, _BB), lambda i: (0, i))
    kern = functools.partial(_tc_step_kernel, decay_prev, apply_update)
    return pl.pallas_call(
        kern,
        grid=(_BH // _BB,),
        in_specs=[
            row_spec,
            row_spec,
            row_spec,
            col_spec,
            idx_spec,
            pl.BlockSpec((_K, _D), lambda i: (0, 0)),
        ],
        out_specs=[idx_spec, idx_spec, col_spec, row_spec, row_spec],
        out_shape=[
            jax.ShapeDtypeStruct((1, _BH), jnp.int32),
            jax.ShapeDtypeStruct((1, _BH), jnp.int32),
            jax.ShapeDtypeStruct((_BH, 1), jnp.float32),
            jax.ShapeDtypeStruct((_BH, _D), jnp.float32),
            jax.ShapeDtypeStruct((_BH, _D), jnp.float32),
        ],
    )(residual, recon, rows, act, bestp, cb_bf16)


def _tc_final_kernel(decay_prev, res_ref, rec_ref, row_ref, act_ref,
                     bestp_ref, sidx_ref, rec_out_ref):
    _, recon, sidx = _finish_prev(
        decay_prev, res_ref, rec_ref, row_ref, act_ref, bestp_ref)
    sidx_ref[0, :] = sidx
    rec_out_ref[...] = recon


def _tc_final(decay_prev, residual, recon, rows, act, bestp):
    row_spec = pl.BlockSpec((_BB, _D), lambda i: (i, 0))
    return pl.pallas_call(
        functools.partial(_tc_final_kernel, decay_prev),
        grid=(_BH // _BB,),
        in_specs=[
            row_spec, row_spec, row_spec,
            pl.BlockSpec((_BB, 1), lambda i: (i, 0)),
            pl.BlockSpec((1, _BB), lambda i: (0, i)),
        ],
        out_specs=[
            pl.BlockSpec((1, _BB), lambda i: (0, i)),
            row_spec,
        ],
        out_shape=[
            jax.ShapeDtypeStruct((1, _BH), jnp.int32),
            jax.ShapeDtypeStruct((_BH, _D), jnp.float32),
        ],
    )(residual, recon, rows, act, bestp)


@functools.cache
def _make_sc_gather():
    nw = _SC_NUM_CORES * _SC_NUM_SUBCORES
    b_per_w = _BH // nw
    mesh = plsc.VectorSubcoreMesh(core_axis_name="c", subcore_axis_name="s",
                                  num_cores=_SC_NUM_CORES)

    @functools.partial(
        pl.kernel, mesh=mesh,
        out_type=jax.ShapeDtypeStruct((_BH, _D), jnp.float32),
        scratch_types=[
            pltpu.VMEM((b_per_w,), jnp.int32),
            pltpu.VMEM((b_per_w, _D), jnp.float32),
            pltpu.SemaphoreType.DMA,
        ],
    )
    def gather(table_hbm, idx_hbm, out_hbm, idx_v, rows_v, sem):
        wid = lax.axis_index("s") * _SC_NUM_CORES + lax.axis_index("c")
        base = wid * b_per_w
        pltpu.sync_copy(idx_hbm.at[pl.ds(base, b_per_w)], idx_v)
        pltpu.async_copy(table_hbm.at[idx_v], rows_v, sem).wait()
        pltpu.sync_copy(rows_v, out_hbm.at[pl.ds(base, b_per_w)])

    return gather


def _sc_gather(table, idx):
    return _make_sc_gather()(table, idx)


@jax.jit
def kernel(targets, codebook):
    cb_bf16 = codebook.astype(jnp.bfloat16)
    halves = []
    for h in range(2):
        halves.append({
            "residual": targets[h * _BH:(h + 1) * _BH],
            "recon": jnp.zeros((_BH, _D), jnp.float32),
            "rows": jnp.zeros((_BH, _D), jnp.float32),
            "act": jnp.zeros((_BH, 1), jnp.float32),
            "best": jnp.zeros((1, _BH), jnp.int32),
            "idx_steps": [],
        })
    for step in range(_L):
        decay_prev = _DECAY ** step
        for st in halves:
            (st["best"], sidx_prev, st["act"], st["residual"],
             st["recon"]) = _tc_step(
                decay_prev, step > 0, st["residual"], st["recon"], st["rows"],
                st["act"], st["best"], cb_bf16)
            if step > 0:
                st["idx_steps"].append(sidx_prev[0])
            st["rows"] = _sc_gather(codebook, st["best"][0])
    decay_last = _DECAY ** _L
    recons = []
    for st in halves:
        sidx_last, recon = _tc_final(
            decay_last, st["residual"], st["recon"], st["rows"], st["act"],
            st["best"])
        st["idx_steps"].append(sidx_last[0])
        recons.append(recon)
    recon = jnp.concatenate(recons, axis=0)
    signed_indices = jnp.concatenate(
        [jnp.stack(st["idx_steps"], axis=1) for st in halves], axis=0)
    return signed_indices, recon


# speedup vs baseline: 1.3629x; 1.0422x over previous
import functools

import jax
import jax.numpy as jnp
from jax import lax
from jax.experimental import pallas as pl
from jax.experimental.pallas import tpu as pltpu
from jax.experimental.pallas import tpu_sc as plsc

_K = 8192
_D = 256
_B = 1024
_L = 16
_DECAY = 0.9
_THRESH = 1e-4

_BB = 256
_NH = 1
_BH = _B // _NH

_SC_NUM_CORES = 2
_SC_NUM_SUBCORES = 16


def _finish_prev(decay_prev, res_ref, rec_ref, row_ref, act_ref, best_ref):
    residual = res_ref[...]
    recon = rec_ref[...]
    rows = row_ref[...]
    d = jnp.sum(residual * rows, axis=1)
    sign = jnp.where(d >= 0, 1.0, -1.0)
    bestp = best_ref[0, :]
    sidx = jnp.where(d >= 0, bestp, -(bestp + 1))
    w = (act_ref[..., 0] * sign) * decay_prev
    contribution = w[:, None] * rows
    return residual - contribution, recon + contribution, sidx


def _tc_step_kernel(decay_prev, apply_update, res_ref, rec_ref, row_ref,
                    act_ref, bestp_ref, cb_ref, best_ref, sidx_ref, act_out_ref,
                    res_out_ref, rec_out_ref):
    if apply_update:
        residual, recon, sidx = _finish_prev(
            decay_prev, res_ref, rec_ref, row_ref, act_ref, bestp_ref)
        sidx_ref[0, :] = sidx
    else:
        residual = res_ref[...]
        recon = rec_ref[...]
        sidx_ref[0, :] = jnp.zeros((res_ref.shape[0],), jnp.int32)
    rn = jnp.sqrt(jnp.sum(residual * residual, axis=1, keepdims=True))
    active = (rn > _THRESH).astype(jnp.float32)
    rnorm = residual / jnp.maximum(rn, 1e-8)
    cos = lax.dot_general(
        rnorm.astype(jnp.bfloat16), cb_ref[...], (((1,), (1,)), ((), ())),
        preferred_element_type=jnp.float32,
        precision=lax.Precision.DEFAULT)
    best = jnp.argmax(jnp.abs(cos), axis=1).astype(jnp.int32)
    best_ref[0, :] = best
    act_out_ref[...] = active
    res_out_ref[...] = residual
    rec_out_ref[...] = recon


def _tc_step(decay_prev, apply_update, residual, recon, rows, act, bestp,
             cb_bf16):
    row_spec = pl.BlockSpec((_BB, _D), lambda i: (i, 0))
    col_spec = pl.BlockSpec((_BB, 1), lambda i: (i, 0))
    idx_spec = pl.BlockSpec((1, _BB), lambda i: (0, i))
    kern = functools.partial(_tc_step_kernel, decay_prev, apply_update)
    return pl.pallas_call(
        kern,
        grid=(_BH // _BB,),
        in_specs=[
            row_spec,
            row_spec,
            row_spec,
            col_spec,
            idx_spec,
            pl.BlockSpec((_K, _D), lambda i: (0, 0)),
        ],
        out_specs=[idx_spec, idx_spec, col_spec, row_spec, row_spec],
        out_shape=[
            jax.ShapeDtypeStruct((1, _BH), jnp.int32),
            jax.ShapeDtypeStruct((1, _BH), jnp.int32),
            jax.ShapeDtypeStruct((_BH, 1), jnp.float32),
            jax.ShapeDtypeStruct((_BH, _D), jnp.float32),
            jax.ShapeDtypeStruct((_BH, _D), jnp.float32),
        ],
    )(residual, recon, rows, act, bestp, cb_bf16)


def _tc_final_kernel(decay_prev, res_ref, rec_ref, row_ref, act_ref,
                     bestp_ref, sidx_ref, rec_out_ref):
    _, recon, sidx = _finish_prev(
        decay_prev, res_ref, rec_ref, row_ref, act_ref, bestp_ref)
    sidx_ref[0, :] = sidx
    rec_out_ref[...] = recon


def _tc_final(decay_prev, residual, recon, rows, act, bestp):
    row_spec = pl.BlockSpec((_BB, _D), lambda i: (i, 0))
    return pl.pallas_call(
        functools.partial(_tc_final_kernel, decay_prev),
        grid=(_BH // _BB,),
        in_specs=[
            row_spec, row_spec, row_spec,
            pl.BlockSpec((_BB, 1), lambda i: (i, 0)),
            pl.BlockSpec((1, _BB), lambda i: (0, i)),
        ],
        out_specs=[
            pl.BlockSpec((1, _BB), lambda i: (0, i)),
            row_spec,
        ],
        out_shape=[
            jax.ShapeDtypeStruct((1, _BH), jnp.int32),
            jax.ShapeDtypeStruct((_BH, _D), jnp.float32),
        ],
    )(residual, recon, rows, act, bestp)


@functools.cache
def _make_sc_gather():
    nw = _SC_NUM_CORES * _SC_NUM_SUBCORES
    b_per_w = _BH // nw
    mesh = plsc.VectorSubcoreMesh(core_axis_name="c", subcore_axis_name="s",
                                  num_cores=_SC_NUM_CORES)

    @functools.partial(
        pl.kernel, mesh=mesh,
        out_type=jax.ShapeDtypeStruct((_BH, _D), jnp.float32),
        scratch_types=[
            pltpu.VMEM((b_per_w,), jnp.int32),
            pltpu.VMEM((b_per_w, _D), jnp.float32),
            pltpu.SemaphoreType.DMA,
        ],
    )
    def gather(table_hbm, idx_hbm, out_hbm, idx_v, rows_v, sem):
        wid = lax.axis_index("s") * _SC_NUM_CORES + lax.axis_index("c")
        base = wid * b_per_w
        pltpu.sync_copy(idx_hbm.at[pl.ds(base, b_per_w)], idx_v)
        pltpu.async_copy(table_hbm.at[idx_v], rows_v, sem).wait()
        pltpu.sync_copy(rows_v, out_hbm.at[pl.ds(base, b_per_w)])

    return gather


def _sc_gather(table, idx):
    return _make_sc_gather()(table, idx)


@jax.jit
def kernel(targets, codebook):
    cb_bf16 = codebook.astype(jnp.bfloat16)
    halves = []
    for h in range(_NH):
        halves.append({
            "residual": targets[h * _BH:(h + 1) * _BH],
            "recon": jnp.zeros((_BH, _D), jnp.float32),
            "rows": jnp.zeros((_BH, _D), jnp.float32),
            "act": jnp.zeros((_BH, 1), jnp.float32),
            "best": jnp.zeros((1, _BH), jnp.int32),
            "idx_steps": [],
        })
    for step in range(_L):
        decay_prev = _DECAY ** step
        for st in halves:
            (st["best"], sidx_prev, st["act"], st["residual"],
             st["recon"]) = _tc_step(
                decay_prev, step > 0, st["residual"], st["recon"], st["rows"],
                st["act"], st["best"], cb_bf16)
            if step > 0:
                st["idx_steps"].append(sidx_prev[0])
            st["rows"] = _sc_gather(codebook, st["best"][0])
    decay_last = _DECAY ** _L
    recons = []
    for st in halves:
        sidx_last, recon = _tc_final(
            decay_last, st["residual"], st["recon"], st["rows"], st["act"],
            st["best"])
        st["idx_steps"].append(sidx_last[0])
        recons.append(recon)
    recon = jnp.concatenate(recons, axis=0)
    signed_indices = jnp.concatenate(
        [jnp.stack(st["idx_steps"], axis=1) for st in halves], axis=0)
    return signed_indices, recon
